# speedup vs baseline: 4.6183x; 4.6183x over previous
import jax
import jax.numpy as jnp
from jax.experimental import pallas as pl

B, S, N, M, W = 16, 128, 4, 2048, 1
BM = 512


def _body(h_ref, out_ref):
    j = pl.program_id(1)
    fp = h_ref[0] & (M - 1)
    rows = jax.lax.broadcasted_iota(jnp.int32, (BM, S), 0) + j * BM
    acc = jnp.zeros((BM, S), jnp.float32)
    for n in range(N):
        acc += (rows == fp[n][None, :]).astype(jnp.float32)
    zero_col = jnp.zeros((BM, 1), jnp.float32)
    right = jnp.concatenate([zero_col, acc[:, : S - 1]], axis=1)
    left = jnp.concatenate([acc[:, 1:], zero_col], axis=1)
    out_ref[0, 0] = right
    out_ref[0, 1] = acc
    out_ref[0, 2] = left


def kernel(sentencesMinHashes):
    h = jnp.transpose(sentencesMinHashes, (0, 2, 1))
    out = pl.pallas_call(
        _body,
        grid=(B, M // BM),
        in_specs=[pl.BlockSpec((1, N, S), lambda b, j: (b, 0, 0))],
        out_specs=pl.BlockSpec((1, 3, BM, S), lambda b, j: (b, 0, j, 0)),
        out_shape=jax.ShapeDtypeStruct((B, 3, M, S), jnp.float32),
    )(h)
    return out.reshape(B, 3 * M, S)


# speedup vs baseline: 8.3702x; 1.8124x over previous
import jax
import jax.numpy as jnp
from jax.experimental import pallas as pl

B, S, N, M, W = 16, 128, 4, 2048, 1
BM = 2048


def _body(h_ref, out_ref):
    j = pl.program_id(1)
    fp = h_ref[0] & (M - 1)
    rows = jax.lax.broadcasted_iota(jnp.int32, (BM, S), 0) + j * BM
    acc = jnp.zeros((BM, S), jnp.float32)
    for n in range(N):
        acc += (rows == fp[n][None, :]).astype(jnp.float32)
    zero_col = jnp.zeros((BM, 1), jnp.float32)
    right = jnp.concatenate([zero_col, acc[:, : S - 1]], axis=1)
    left = jnp.concatenate([acc[:, 1:], zero_col], axis=1)
    out_ref[0, 0] = right
    out_ref[0, 1] = acc
    out_ref[0, 2] = left


def kernel(sentencesMinHashes):
    h = jnp.transpose(sentencesMinHashes, (0, 2, 1))
    out = pl.pallas_call(
        _body,
        grid=(B, M // BM),
        in_specs=[pl.BlockSpec((1, N, S), lambda b, j: (b, 0, 0))],
        out_specs=pl.BlockSpec((1, 3, BM, S), lambda b, j: (b, 0, j, 0)),
        out_shape=jax.ShapeDtypeStruct((B, 3, M, S), jnp.float32),
    )(h)
    return out.reshape(B, 3 * M, S)


# speedup vs baseline: 9.5134x; 1.1366x over previous
import jax
import jax.numpy as jnp
from jax.experimental import pallas as pl

B, S, N, M, W = 16, 128, 4, 2048, 1
BB = 2


def _body(h_ref, out_ref):
    rows = jax.lax.broadcasted_iota(jnp.int32, (M, S), 0)
    zero_col = jnp.zeros((M, 1), jnp.float32)
    for b in range(BB):
        fp = h_ref[b] & (M - 1)
        acc = jnp.zeros((M, S), jnp.float32)
        for n in range(N):
            acc += (rows == fp[n][None, :]).astype(jnp.float32)
        right = jnp.concatenate([zero_col, acc[:, : S - 1]], axis=1)
        left = jnp.concatenate([acc[:, 1:], zero_col], axis=1)
        out_ref[b, 0] = right
        out_ref[b, 1] = acc
        out_ref[b, 2] = left


def kernel(sentencesMinHashes):
    h = jnp.transpose(sentencesMinHashes, (0, 2, 1))
    out = pl.pallas_call(
        _body,
        grid=(B // BB,),
        in_specs=[pl.BlockSpec((BB, N, S), lambda b: (b, 0, 0))],
        out_specs=pl.BlockSpec((BB, 3, M, S), lambda b: (b, 0, 0, 0)),
        out_shape=jax.ShapeDtypeStruct((B, 3, M, S), jnp.float32),
    )(h)
    return out.reshape(B, 3 * M, S)
